# baseline (speedup 1.0000x reference)
import functools

import jax
import jax.numpy as jnp
from jax import lax
from jax.experimental import pallas as pl
from jax.experimental.pallas import tpu as pltpu
from jax.experimental.pallas import tpu_sc as plsc

_IPNUM = 8192
_B = 16384
_D = 8
_NC = 2
_NS = 16
_NW = _NC * _NS
_BPW = _B // _NW
_CH = 128
_NCH = _BPW // _CH
_L = 16

_BS = 16384


def _make_sc_gather():
    mesh = plsc.VectorSubcoreMesh(core_axis_name="c", subcore_axis_name="s")

    @functools.partial(
        pl.kernel,
        mesh=mesh,
        compiler_params=pltpu.CompilerParams(use_tc_tiling_on_sc=False),
        out_type=jax.ShapeDtypeStruct((7, _B), jnp.float32),
        scratch_types=[
            pltpu.VMEM((_NCH, _CH), jnp.int32),
            pltpu.VMEM((5 * _NCH, _CH), jnp.int32),
            pltpu.VMEM((5, _BPW), jnp.float32),
            pltpu.SemaphoreType.DMA,
        ],
    )
    def sc_gather(tflat_hbm, idx_hbm, time_hbm, t1f_hbm, out_hbm,
                  idx_v, idxc_v, rt_v, sem):
        wid = lax.axis_index("s") * _NC + lax.axis_index("c")
        base = wid * _BPW
        pltpu.sync_copy(idx_hbm.at[wid], idx_v)
        pltpu.sync_copy(time_hbm.at[0, pl.ds(base, _BPW)],
                        out_hbm.at[5, pl.ds(base, _BPW)])
        pltpu.sync_copy(t1f_hbm.at[0, pl.ds(base, _BPW)],
                        out_hbm.at[6, pl.ds(base, _BPW)])
        for j in range(_NCH):
            for k in range(_CH // _L):
                v = idx_v[j, pl.ds(k * _L, _L)]
                v5 = v * 5
                for c in range(5):
                    idxc_v[c * _NCH + j, pl.ds(k * _L, _L)] = v5 + c
        copies = [
            pltpu.async_copy(
                tflat_hbm.at[idxc_v.at[c * _NCH + j]],
                rt_v.at[c, pl.ds(j * _CH, _CH)],
                sem,
            )
            for c in range(5)
            for j in range(_NCH)
        ]
        for cp in copies:
            cp.wait()
        for c in range(5):
            pltpu.sync_copy(rt_v.at[c], out_hbm.at[c, pl.ds(base, _BPW)])

    return sc_gather


def _mlp_body(rows_ref, e2t_ref, w1a_ref, w1b_ref,
              w1t_ref, b1_ref, w2t_ref, b2_ref, w3t_ref, b3_ref,
              w4t_ref, b4_ref, out_ref):
    f32 = jnp.float32
    rows = rows_ref[...]
    xT = rows[0:5, :]
    timeT = rows[5:6, :]
    t1i = lax.bitcast_convert_type(rows[6:7, :], jnp.int32)
    h = jnp.dot(w1a_ref[...], xT, preferred_element_type=f32)
    e2wT = jnp.dot(w1b_ref[...], e2t_ref[...],
                   preferred_element_type=f32)
    ohT = (lax.broadcasted_iota(jnp.int32, (21, 1), 0) == t1i
           ).astype(f32)
    h = h + jnp.dot(e2wT, ohT, preferred_element_type=f32)
    h = h + jnp.dot(w1t_ref[...], timeT, preferred_element_type=f32)
    x = jnp.tanh(h + b1_ref[...])
    x = jnp.tanh(jnp.dot(w2t_ref[...], x, preferred_element_type=f32)
                 + b2_ref[...])
    x = jnp.tanh(jnp.dot(w3t_ref[...], x, preferred_element_type=f32)
                 + b3_ref[...])
    x = jnp.tanh(jnp.dot(w4t_ref[...], x, preferred_element_type=f32)
                 + b4_ref[...])
    out_ref[...] = jax.nn.sigmoid(x)


def kernel(ipa, type1, time, emb1, emb2, W1, b1, W2, b2, W3, b3, W4, b4):
    f32 = jnp.float32
    idx = ipa.reshape(_NW, _NCH, _CH)
    timeT = time.reshape(1, _B)
    t1f = lax.bitcast_convert_type(type1, f32).reshape(1, _B)
    rowsT = _make_sc_gather()(emb1.reshape(-1), idx, timeT, t1f)

    w1a = W1[:5].T
    w1b = W1[5:10].T
    w1t = W1[10:11].T

    full = lambda a, b: pl.BlockSpec((a, b), lambda i: (0, 0))
    out = pl.pallas_call(
        _mlp_body,
        grid=(_B // _BS,),
        in_specs=[
            pl.BlockSpec((7, _BS), lambda i: (0, i)),
            full(5, 21),
            full(20, 5), full(20, 5), full(20, 1), full(20, 1),
            full(30, 20), full(30, 1),
            full(10, 30), full(10, 1),
            full(1, 10), full(1, 1),
        ],
        out_specs=pl.BlockSpec((1, _BS), lambda i: (0, i)),
        out_shape=jax.ShapeDtypeStruct((1, _B), f32),
    )(rowsT, emb2.T, w1a, w1b, w1t, b1.reshape(20, 1),
      W2.T, b2.reshape(30, 1), W3.T, b3.reshape(10, 1),
      W4.T, b4.reshape(1, 1))
    return out.reshape(_B, 1)


# speedup vs baseline: 1.1428x; 1.1428x over previous
import functools

import jax
import jax.numpy as jnp
from jax import lax
from jax.experimental import pallas as pl
from jax.experimental.pallas import tpu as pltpu
from jax.experimental.pallas import tpu_sc as plsc

_IPNUM = 8192
_B = 16384
_D = 8
_NC = 2
_NS = 16
_NW = _NC * _NS
_BPW = _B // _NW
_CH = 128
_NCH = _BPW // _CH
_L = 16

_BS = 16384


def _make_sc_gather():
    mesh = plsc.VectorSubcoreMesh(core_axis_name="c", subcore_axis_name="s")

    @functools.partial(
        pl.kernel,
        mesh=mesh,
        compiler_params=pltpu.CompilerParams(use_tc_tiling_on_sc=False),
        out_type=jax.ShapeDtypeStruct((5, _B), jnp.float32),
        scratch_types=[
            pltpu.VMEM((_NCH, _CH), jnp.int32),
            pltpu.VMEM((5 * _NCH, _CH), jnp.int32),
            pltpu.VMEM((5, _BPW), jnp.float32),
            pltpu.SemaphoreType.DMA,
        ],
    )
    def sc_gather(tflat_hbm, idx_hbm, out_hbm, idx_v, idxc_v, rt_v, sem):
        wid = lax.axis_index("s") * _NC + lax.axis_index("c")
        base = wid * _BPW
        pltpu.sync_copy(idx_hbm.at[wid], idx_v)
        for j in range(_NCH):
            for k in range(_CH // _L):
                v = idx_v[j, pl.ds(k * _L, _L)]
                v5 = v * 5
                for c in range(5):
                    idxc_v[c * _NCH + j, pl.ds(k * _L, _L)] = v5 + c
        copies = [
            pltpu.async_copy(
                tflat_hbm.at[idxc_v.at[c * _NCH + j]],
                rt_v.at[c, pl.ds(j * _CH, _CH)],
                sem,
            )
            for c in range(5)
            for j in range(_NCH)
        ]
        for cp in copies:
            cp.wait()
        for c in range(5):
            pltpu.sync_copy(rt_v.at[c], out_hbm.at[c, pl.ds(base, _BPW)])

    return sc_gather


def _mlp_body(rows_ref, t1_ref, time_ref, e2t_ref, w1a_ref, w1b_ref,
              w1t_ref, b1_ref, w2t_ref, b2_ref, w3t_ref, b3_ref,
              w4t_ref, b4_ref, out_ref):
    f32 = jnp.float32
    xT = rows_ref[...]
    h = jnp.dot(w1a_ref[...], xT, preferred_element_type=f32)
    e2wT = jnp.dot(w1b_ref[...], e2t_ref[...],
                   preferred_element_type=f32)
    ohT = (lax.broadcasted_iota(jnp.int32, (21, 1), 0) == t1_ref[...]
           ).astype(f32)
    h = h + jnp.dot(e2wT, ohT, preferred_element_type=f32)
    h = h + jnp.dot(w1t_ref[...], time_ref[...], preferred_element_type=f32)
    x = jnp.tanh(h + b1_ref[...])
    x = jnp.tanh(jnp.dot(w2t_ref[...], x, preferred_element_type=f32)
                 + b2_ref[...])
    x = jnp.tanh(jnp.dot(w3t_ref[...], x, preferred_element_type=f32)
                 + b3_ref[...])
    x = jnp.tanh(jnp.dot(w4t_ref[...], x, preferred_element_type=f32)
                 + b4_ref[...])
    out_ref[...] = jax.nn.sigmoid(x)


def kernel(ipa, type1, time, emb1, emb2, W1, b1, W2, b2, W3, b3, W4, b4):
    f32 = jnp.float32
    idx = ipa.reshape(_NW, _NCH, _CH)
    rowsT = _make_sc_gather()(emb1.reshape(-1), idx)

    t1T = type1.reshape(1, _B)
    timeT = time.reshape(1, _B)
    w1a = W1[:5].T
    w1b = W1[5:10].T
    w1t = W1[10:11].T

    full = lambda a, b: pl.BlockSpec((a, b), lambda i: (0, 0))
    out = pl.pallas_call(
        _mlp_body,
        grid=(_B // _BS,),
        in_specs=[
            pl.BlockSpec((5, _BS), lambda i: (0, i)),
            pl.BlockSpec((1, _BS), lambda i: (0, i)),
            pl.BlockSpec((1, _BS), lambda i: (0, i)),
            full(5, 21),
            full(20, 5), full(20, 5), full(20, 1), full(20, 1),
            full(30, 20), full(30, 1),
            full(10, 30), full(10, 1),
            full(1, 10), full(1, 1),
        ],
        out_specs=pl.BlockSpec((1, _BS), lambda i: (0, i)),
        out_shape=jax.ShapeDtypeStruct((1, _B), f32),
    )(rowsT, t1T, timeT, emb2.T, w1a, w1b, w1t, b1.reshape(20, 1),
      W2.T, b2.reshape(30, 1), W3.T, b3.reshape(10, 1),
      W4.T, b4.reshape(1, 1))
    return out.reshape(_B, 1)
